# baseline (speedup 1.0000x reference)
import functools

import jax
import jax.numpy as jnp
from jax import lax
from jax.experimental import pallas as pl
from jax.experimental.pallas import tpu as pltpu
from jax.experimental.pallas import tpu_sc as plsc

D = 128
SEQ = 2048
BATCH = 4
B_TOTAL = BATCH * SEQ
NC, NS, L = 2, 16, 16
NW = NC * NS
BPW = B_TOTAL // NW
SCH = SEQ // NW


HCH = SCH // 2


def _emb_body(idx_hbm, word_hbm, pos_hbm, out_hbm, idx_v, rows_v, pos_v,
              isem, gsem, psem, osem):
    wid = lax.axis_index("s") * NC + lax.axis_index("c")
    s0 = wid * SCH

    idxcp = [
        pltpu.async_copy(idx_hbm.at[pl.ds(b * SEQ + s0, SCH)],
                         idx_v.at[b], isem.at[b])
        for b in range(BATCH)
    ]
    poscp = pltpu.async_copy(pos_hbm.at[pl.ds(s0, SCH)], pos_v, psem)

    gathers = []
    for b in range(BATCH):
        idxcp[b].wait()
        gathers.append(
            pltpu.async_copy(word_hbm.at[idx_v.at[b]],
                             rows_v.at[pl.ds(b * SCH, SCH)], gsem.at[b]))

    poscp.wait()
    outs = []
    for b in range(BATCH):
        gathers[b].wait()
        for h in range(2):
            base = b * SCH + h * HCH
            pbase = h * HCH

            def add_rows(j, _, base=base, pbase=pbase):
                for r in (2 * j, 2 * j + 1):
                    for k in range(D // L):
                        sl = pl.ds(k * L, L)
                        plsc.addupdate(rows_v.at[base + r, sl],
                                       pos_v[pbase + r, sl])
                return 0

            lax.fori_loop(0, HCH // 2, add_rows, 0)
            outs.append(
                pltpu.async_copy(rows_v.at[pl.ds(base, HCH)],
                                 out_hbm.at[pl.ds(b * SEQ + s0 + h * HCH,
                                                  HCH)],
                                 osem.at[2 * b + h]))
    for o in outs:
        o.wait()


@jax.jit
def kernel(inputs, word_embedding, position_embedding):
    idx = inputs.astype(jnp.int32).reshape(B_TOTAL)
    mesh = plsc.VectorSubcoreMesh(core_axis_name="c", subcore_axis_name="s")
    out = pl.kernel(
        _emb_body,
        mesh=mesh,
        out_type=jax.ShapeDtypeStruct((B_TOTAL, D), jnp.float32),
        scratch_types=[
            pltpu.VMEM((BATCH, SCH), jnp.int32),
            pltpu.VMEM((BPW, D), jnp.float32),
            pltpu.VMEM((SCH, D), jnp.float32),
            pltpu.SemaphoreType.DMA((BATCH,)),
            pltpu.SemaphoreType.DMA((BATCH,)),
            pltpu.SemaphoreType.DMA,
            pltpu.SemaphoreType.DMA((2 * BATCH,)),
        ],
    )(idx, word_embedding, position_embedding)
    return out.reshape(BATCH, SEQ, D)


# speedup vs baseline: 1.0399x; 1.0399x over previous
import functools

import jax
import jax.numpy as jnp
from jax import lax
from jax.experimental import pallas as pl
from jax.experimental.pallas import tpu as pltpu
from jax.experimental.pallas import tpu_sc as plsc

D = 128
SEQ = 2048
BATCH = 4
B_TOTAL = BATCH * SEQ
NC, NS, L = 2, 16, 16
NW = NC * NS
BPW = B_TOTAL // NW
SCH = SEQ // NW


def _emb_body(idx_hbm, word_hbm, pos_hbm, out_hbm, idx_v, rows_v, pos_v,
              isem, gsem, psem, osem):
    wid = lax.axis_index("s") * NC + lax.axis_index("c")
    s0 = wid * SCH

    idxcp = [
        pltpu.async_copy(idx_hbm.at[pl.ds(b * SEQ + s0, SCH)],
                         idx_v.at[b], isem.at[b])
        for b in range(BATCH)
    ]
    poscp = pltpu.async_copy(pos_hbm.at[pl.ds(s0, SCH)], pos_v, psem)

    gathers = []
    for b in range(BATCH):
        idxcp[b].wait()
        gathers.append(
            pltpu.async_copy(word_hbm.at[idx_v.at[b]],
                             rows_v.at[pl.ds(b * SCH, SCH)], gsem.at[b]))

    poscp.wait()
    outs = []
    for b in range(BATCH):
        gathers[b].wait()

        def add_row(j, _, b=b):
            row = b * SCH + j
            for k in range(D // L):
                sl = pl.ds(k * L, L)
                plsc.addupdate(rows_v.at[row, sl], pos_v[j, sl])
            return 0

        lax.fori_loop(0, SCH, add_row, 0)
        outs.append(
            pltpu.async_copy(rows_v.at[pl.ds(b * SCH, SCH)],
                             out_hbm.at[pl.ds(b * SEQ + s0, SCH)],
                             osem.at[b]))
    for o in outs:
        o.wait()


@jax.jit
def kernel(inputs, word_embedding, position_embedding):
    idx = inputs.astype(jnp.int32).reshape(B_TOTAL)
    mesh = plsc.VectorSubcoreMesh(core_axis_name="c", subcore_axis_name="s")
    out = pl.kernel(
        _emb_body,
        mesh=mesh,
        out_type=jax.ShapeDtypeStruct((B_TOTAL, D), jnp.float32),
        scratch_types=[
            pltpu.VMEM((BATCH, SCH), jnp.int32),
            pltpu.VMEM((BPW, D), jnp.float32),
            pltpu.VMEM((SCH, D), jnp.float32),
            pltpu.SemaphoreType.DMA((BATCH,)),
            pltpu.SemaphoreType.DMA((BATCH,)),
            pltpu.SemaphoreType.DMA,
            pltpu.SemaphoreType.DMA((BATCH,)),
        ],
    )(idx, word_embedding, position_embedding)
    return out.reshape(BATCH, SEQ, D)
